# Initial kernel scaffold; baseline (speedup 1.0000x reference)
import jax
import jax.numpy as jnp
from jax.experimental import pallas as pl


def kernel(electrons, nuclei, feat_same, feat_anti, feat_ne, u_W1, u_b1, u_W2, u_b2, w_W1, w_b1, w_W2, w_b2, h_W1, h_b1, h_W2, h_b2, g_W1, g_b1, g_W2, g_b2, send_same, recv_same, send_anti, recv_anti, send_ne, recv_ne):
    raise NotImplementedError("write your pallas kernel here")


# speedup vs baseline: 2.6682x; 2.6682x over previous
import functools

import jax
import jax.numpy as jnp
from jax import lax
from jax.experimental import pallas as pl
from jax.experimental.pallas import tpu as pltpu
from jax.experimental.pallas import tpu_sc as plsc


def _edge_block_body(f_ref, uw1, ub1, uw2, ub2, ww1, wb1, ww2, wb2,
                     up_ref, we_ref):
    f = f_ref[...]
    t = jnp.tanh(jnp.dot(f, uw1[...], preferred_element_type=jnp.float32)
                 + ub1[...])
    up = f + jnp.dot(t, uw2[...], preferred_element_type=jnp.float32) + ub2[...]
    up_ref[...] = up
    t2 = jnp.tanh(jnp.dot(up, ww1[...], preferred_element_type=jnp.float32)
                  + wb1[...])
    we_ref[...] = (jnp.dot(t2, ww2[...], preferred_element_type=jnp.float32)
                   + wb2[...])


def _edge_mlps(f, uw1, ub1, uw2, ub2, ww1, wb1, ww2, wb2, blk):
    e, d = f.shape
    wspec = pl.BlockSpec((d, d), lambda i: (0, 0))
    bspec = pl.BlockSpec((1, d), lambda i: (0, 0))
    fspec = pl.BlockSpec((blk, d), lambda i: (i, 0))
    return pl.pallas_call(
        _edge_block_body,
        grid=(e // blk,),
        in_specs=[fspec, wspec, bspec, wspec, bspec,
                  wspec, bspec, wspec, bspec],
        out_specs=[fspec, fspec],
        out_shape=[jax.ShapeDtypeStruct((e, d), jnp.float32)] * 2,
    )(f, uw1, ub1.reshape(1, d), uw2, ub2.reshape(1, d),
      ww1, wb1.reshape(1, d), ww2, wb2.reshape(1, d))


def _node_mlp_body(x_ref, w1, b1, w2, b2, o_ref):
    t = jnp.tanh(jnp.dot(x_ref[...], w1[...],
                         preferred_element_type=jnp.float32) + b1[...])
    o_ref[...] = jnp.dot(t, w2[...], preferred_element_type=jnp.float32) + b2[...]


def _node_mlp(x, w1, b1, w2, b2, blk):
    n, d = x.shape
    wspec = pl.BlockSpec((d, d), lambda i: (0, 0))
    bspec = pl.BlockSpec((1, d), lambda i: (0, 0))
    xspec = pl.BlockSpec((blk, d), lambda i: (i, 0))
    return pl.pallas_call(
        _node_mlp_body,
        grid=(n // blk,),
        in_specs=[xspec, wspec, bspec, wspec, bspec],
        out_specs=xspec,
        out_shape=jax.ShapeDtypeStruct((n, d), jnp.float32),
    )(x, w1, b1.reshape(1, d), w2, b2.reshape(1, d))


_NC = 2
_NS = 16
_CH = 128


def _sc_conv(we, h_emb, send, recv, n_out):
    e, d = we.shape
    nw = _NC * _NS
    n_chunks = e // _CH
    assert n_chunks * _CH == e
    per = n_chunks // nw
    extra = n_chunks - per * nw
    zb = 128
    nblk = -(-n_out // zb)
    n_pad = nblk * zb
    blk_per_sub = -(-nblk // _NS)

    mesh = plsc.VectorSubcoreMesh(core_axis_name="c", subcore_axis_name="s")

    @functools.partial(
        pl.kernel,
        out_type=jax.ShapeDtypeStruct((_NC * n_pad, d), jnp.float32),
        mesh=mesh,
        scratch_types=[
            pltpu.VMEM((_CH,), jnp.int32),
            pltpu.VMEM((_CH,), jnp.int32),
            pltpu.VMEM((_CH, d), jnp.float32),
            pltpu.VMEM((_CH, d), jnp.float32),
            pltpu.VMEM((zb, d), jnp.float32),
            pltpu.VMEM_SHARED((n_pad, d), jnp.float32),
            pltpu.SemaphoreType.DMA,
        ],
    )
    def conv(we_hbm, h_hbm, send_hbm, recv_hbm, out_hbm,
             sidx, ridx, wev, hxv, zrow, zacc, sem):
        c = lax.axis_index("c")
        s = lax.axis_index("s")
        wid = s * _NC + c

        zeros16 = jnp.zeros((16,), jnp.float32)

        def zero_row(i, _):
            for j in range(d // 16):
                zrow[i, pl.ds(j * 16, 16)] = zeros16
            return 0

        lax.fori_loop(0, zb, zero_row, 0)

        def zero_stripe(i, _):
            b = s + i * _NS

            @pl.when(b < nblk)
            def _():
                pltpu.sync_copy(zrow, zacc.at[pl.ds(b * zb, zb)])

            return 0

        lax.fori_loop(0, blk_per_sub, zero_stripe, 0)
        plsc.subcore_barrier()

        def chunk(base):
            pltpu.sync_copy(send_hbm.at[pl.ds(base, _CH)], sidx)
            pltpu.async_copy(h_hbm.at[sidx], hxv, sem).wait()
            pltpu.sync_copy(we_hbm.at[pl.ds(base, _CH)], wev)

            def mul(i, _):
                for j in range(d // 16):
                    sl = pl.ds(j * 16, 16)
                    wev[i, sl] = wev[i, sl] * hxv[i, sl]
                return 0

            lax.fori_loop(0, _CH, mul, 0)
            pltpu.sync_copy(recv_hbm.at[pl.ds(base, _CH)], ridx)
            pltpu.sync_copy(wev, zacc.at[ridx], add=True)

        def loop(k, _):
            chunk((wid * per + k) * _CH)
            return 0

        lax.fori_loop(0, per, loop, 0)
        if extra:
            @pl.when(wid < extra)
            def _tail():
                chunk((nw * per + wid) * _CH)

        plsc.subcore_barrier()

        def write_out(i, _):
            b = s + i * _NS

            @pl.when(b < nblk)
            def _():
                off = b * zb
                pltpu.sync_copy(zacc.at[pl.ds(off, zb)],
                                out_hbm.at[pl.ds(c * n_pad + off, zb)])

            return 0

        lax.fori_loop(0, blk_per_sub, write_out, 0)

    out = conv(we, h_emb, send.astype(jnp.int32), recv.astype(jnp.int32))
    return out.reshape(_NC, n_pad, d)


def _final_body(el_ref, z0_ref, z1_ref, z2_ref, gw1, gb1, gw2, gb2, o_ref):
    def mlp(x, j):
        t = jnp.tanh(jnp.dot(x, gw1[j], preferred_element_type=jnp.float32)
                     + gb1[j])
        return jnp.dot(t, gw2[j], preferred_element_type=jnp.float32) + gb2[j]

    el = el_ref[...]
    acc = el + mlp(el, 0)
    acc = acc + mlp(z0_ref[0] + z0_ref[1], 1)
    acc = acc + mlp(z1_ref[0] + z1_ref[1], 2)
    acc = acc + mlp(z2_ref[0] + z2_ref[1], 3)
    o_ref[...] = acc


def _final_update(electrons, z0, z1, z2, gw1, gb1, gw2, gb2, blk):
    n, d = electrons.shape
    espec = pl.BlockSpec((blk, d), lambda i: (i, 0))
    zspec = pl.BlockSpec((_NC, blk, d), lambda i: (0, i, 0))
    return pl.pallas_call(
        _final_body,
        grid=(n // blk,),
        in_specs=[espec, zspec, zspec, zspec,
                  pl.BlockSpec((4, d, d), lambda i: (0, 0, 0)),
                  pl.BlockSpec((4, d), lambda i: (0, 0)),
                  pl.BlockSpec((4, d, d), lambda i: (0, 0, 0)),
                  pl.BlockSpec((4, d), lambda i: (0, 0))],
        out_specs=espec,
        out_shape=jax.ShapeDtypeStruct((n, d), jnp.float32),
    )(electrons, z0, z1, z2, gw1, gb1, gw2, gb2)


def kernel(electrons, nuclei, feat_same, feat_anti, feat_ne,
           u_W1, u_b1, u_W2, u_b2, w_W1, w_b1, w_W2, w_b2,
           h_W1, h_b1, h_W2, h_b2, g_W1, g_b1, g_W2, g_b2,
           send_same, recv_same, send_anti, recv_anti, send_ne, recv_ne):
    n_el, d = electrons.shape

    feats = (feat_same, feat_anti, feat_ne)
    ups, wes = [], []
    for i in range(3):
        up, we = _edge_mlps(feats[i], u_W1[i], u_b1[i], u_W2[i], u_b2[i],
                            w_W1[i], w_b1[i], w_W2[i], w_b2[i], blk=2000)
        ups.append(up)
        wes.append(we)

    h0 = _node_mlp(electrons, h_W1[0], h_b1[0], h_W2[0], h_b2[0], blk=2000)
    h1 = _node_mlp(electrons, h_W1[1], h_b1[1], h_W2[1], h_b2[1], blk=2000)
    h2 = _node_mlp(nuclei, h_W1[2], h_b1[2], h_W2[2], h_b2[2], blk=1000)

    z0 = _sc_conv(wes[0], h0, send_same, recv_same, n_el)
    z1 = _sc_conv(wes[1], h1, send_anti, recv_anti, n_el)
    z2 = _sc_conv(wes[2], h2, send_ne, recv_ne, n_el)

    updated = _final_update(electrons, z0, z1, z2,
                            g_W1, g_b1, g_W2, g_b2, blk=2000)
    return (updated, ups[0], ups[1], ups[2])
